# baseline (speedup 1.0000x reference)
import functools
import jax
import jax.numpy as jnp
from jax import lax
from jax.experimental import pallas as pl
from jax.experimental.pallas import tpu as pltpu
from jax.experimental.pallas import tpu_sc as plsc

EMBED = 64
SCALE = 8.0
CHUNK = 128
NBUF = 4


def _sc_embed(table, idx3):
    nw, n_chunks, _ = idx3.shape
    b_per_w = n_chunks * CHUNK
    B = nw * b_per_w

    mesh = plsc.VectorSubcoreMesh(core_axis_name="c", subcore_axis_name="s")
    info = plsc.get_sparse_core_info()
    nc = info.num_cores

    @functools.partial(
        pl.kernel,
        out_type=jax.ShapeDtypeStruct((B, EMBED), jnp.float32),
        mesh=mesh,
        compiler_params=pltpu.CompilerParams(use_tc_tiling_on_sc=False),
        scratch_types=[
            pltpu.VMEM((n_chunks, CHUNK), jnp.int32),
            [pltpu.VMEM((CHUNK, EMBED), jnp.float32) for _ in range(NBUF)],
            [pltpu.SemaphoreType.DMA for _ in range(NBUF)],
            [pltpu.SemaphoreType.DMA for _ in range(NBUF)],
        ],
    )
    def k(table_hbm, idx_hbm, out_hbm, idx_v, bufs, gsems, wsems):
        wid = lax.axis_index("s") * nc + lax.axis_index("c")
        base = wid * b_per_w
        pltpu.sync_copy(idx_hbm.at[wid], idx_v)

        def start_gather(i, slot):
            pltpu.async_copy(
                table_hbm.at[idx_v.at[i]], bufs[slot], gsems[slot]
            )

        def wait_gather(slot):
            pltpu.make_async_copy(
                table_hbm.at[idx_v.at[0]], bufs[slot], gsems[slot]
            ).wait()

        def start_write(i, slot):
            pltpu.async_copy(
                bufs[slot], out_hbm.at[pl.ds(base + i * CHUNK, CHUNK)],
                wsems[slot],
            )

        def wait_write(slot):
            pltpu.make_async_copy(
                bufs[slot], out_hbm.at[pl.ds(base, CHUNK)], wsems[slot]
            ).wait()

        def scale(slot):
            buf = bufs[slot]

            def row(r, _):
                for j in range(EMBED // 16):
                    sl = pl.ds(16 * j, 16)
                    buf[r, sl] = buf[r, sl] * SCALE
                return _

            lax.fori_loop(0, CHUNK, row, None, unroll=2)

        start_gather(0, 0)
        start_gather(1, 1)

        def pair(p, _):
            for b in range(NBUF):
                i = p * NBUF + b
                nslot = (b + 2) % NBUF

                @pl.when(i + 2 < n_chunks)
                def _():
                    @pl.when(i >= 2)
                    def _():
                        wait_write(nslot)

                    start_gather(i + 2, nslot)

                wait_gather(b)
                scale(b)
                start_write(i, b)
            return _

        lax.fori_loop(0, n_chunks // NBUF, pair, None)
        for b in range(NBUF):
            wait_write(b)

    return k(table, idx3)


def kernel(inputs, table):
    nb, nt = inputs.shape
    B = nb * nt
    info = plsc.get_sparse_core_info()
    nw = info.num_cores * info.num_subcores
    n_chunks = B // (nw * CHUNK)
    idx3 = inputs.reshape(nw, n_chunks, CHUNK)
    out = _sc_embed(table, idx3)
    return out.reshape(nb, nt, EMBED)


# speedup vs baseline: 1.0009x; 1.0009x over previous
import functools
import jax
import jax.numpy as jnp
from jax import lax
from jax.experimental import pallas as pl
from jax.experimental.pallas import tpu as pltpu
from jax.experimental.pallas import tpu_sc as plsc

EMBED = 64
SCALE = 8.0
CHUNK = 128
NBUF = 4


def _sc_embed(table, idx3):
    nw, n_chunks, _ = idx3.shape
    b_per_w = n_chunks * CHUNK
    B = nw * b_per_w

    mesh = plsc.VectorSubcoreMesh(core_axis_name="c", subcore_axis_name="s")
    info = plsc.get_sparse_core_info()
    nc = info.num_cores

    @functools.partial(
        pl.kernel,
        out_type=jax.ShapeDtypeStruct((B, EMBED), jnp.float32),
        mesh=mesh,
        compiler_params=pltpu.CompilerParams(use_tc_tiling_on_sc=False),
        scratch_types=[
            pltpu.VMEM((n_chunks, CHUNK), jnp.int32),
            [pltpu.VMEM((CHUNK, EMBED), jnp.float32) for _ in range(NBUF)],
            [pltpu.SemaphoreType.DMA for _ in range(NBUF)],
            [pltpu.SemaphoreType.DMA for _ in range(NBUF)],
        ],
    )
    def k(table_hbm, idx_hbm, out_hbm, idx_v, bufs, gsems, wsems):
        wid = lax.axis_index("s") * nc + lax.axis_index("c")
        base = wid * b_per_w
        pltpu.sync_copy(idx_hbm.at[wid], idx_v)

        def start_gather(i, slot):
            pltpu.async_copy(
                table_hbm.at[idx_v.at[i]], bufs[slot], gsems[slot]
            )

        def wait_gather(slot):
            pltpu.make_async_copy(
                table_hbm.at[idx_v.at[0]], bufs[slot], gsems[slot]
            ).wait()

        def start_write(i, slot):
            pltpu.async_copy(
                bufs[slot], out_hbm.at[pl.ds(base + i * CHUNK, CHUNK)],
                wsems[slot],
            )

        def wait_write(slot):
            pltpu.make_async_copy(
                bufs[slot], out_hbm.at[pl.ds(base, CHUNK)], wsems[slot]
            ).wait()

        def scale(slot):
            buf = bufs[slot]

            @plsc.parallel_loop(0, CHUNK, unroll=4)
            def _(r):
                for j in range(EMBED // 16):
                    sl = pl.ds(16 * j, 16)
                    buf[r, sl] = buf[r, sl] * SCALE

        start_gather(0, 0)
        start_gather(1, 1)

        def pair(p, _):
            for b in range(NBUF):
                i = p * NBUF + b
                nslot = (b + 2) % NBUF

                @pl.when(i + 2 < n_chunks)
                def _():
                    @pl.when(i >= 2)
                    def _():
                        wait_write(nslot)

                    start_gather(i + 2, nslot)

                wait_gather(b)
                scale(b)
                start_write(i, b)
            return _

        lax.fori_loop(0, n_chunks // NBUF, pair, None)
        for b in range(NBUF):
            wait_write(b)

    return k(table, idx3)


def kernel(inputs, table):
    nb, nt = inputs.shape
    B = nb * nt
    info = plsc.get_sparse_core_info()
    nw = info.num_cores * info.num_subcores
    n_chunks = B // (nw * CHUNK)
    idx3 = inputs.reshape(nw, n_chunks, CHUNK)
    out = _sc_embed(table, idx3)
    return out.reshape(nb, nt, EMBED)
